# Initial kernel scaffold; baseline (speedup 1.0000x reference)
import jax
import jax.numpy as jnp
from jax.experimental import pallas as pl


def kernel(x, edge_index, W0, b0, W1, b1, W2, b2, W3, b3):
    raise NotImplementedError("write your pallas kernel here")


# speedup vs baseline: 5.3379x; 5.3379x over previous
import functools

import jax
import jax.numpy as jnp
from jax import lax
from jax.experimental import pallas as pl
from jax.experimental.pallas import tpu as pltpu
from jax.experimental.pallas import tpu_sc as plsc

N = 10000
E = 320000
D = 128

NUM_CORES = 2
NUM_SUBCORES = 16
NUM_WORKERS = NUM_CORES * NUM_SUBCORES
E_PER_W = E // NUM_WORKERS
CHUNK = 80
NCHUNK = E_PER_W // CHUNK

ROWS_PER_SUB = 624
ROW_TAIL = N - NUM_SUBCORES * ROWS_PER_SUB

_SELU_ALPHA = 1.6732632423543772
_SELU_SCALE = 1.0507009873554805


def _selu(h):
    neg = _SELU_ALPHA * (jnp.exp(jnp.minimum(h, 0.0)) - 1.0)
    return _SELU_SCALE * jnp.where(h > 0, h, neg)


def _make_agg(with_deg):
    mesh = plsc.VectorSubcoreMesh(core_axis_name="core", subcore_axis_name="sub")
    out_type = [jax.ShapeDtypeStruct((NUM_CORES, N, D), jnp.float32)]
    if with_deg:
        out_type.append(jax.ShapeDtypeStruct((NUM_CORES * N,), jnp.float32))
    scratch_types = [
        pltpu.VMEM_SHARED((N, D), jnp.float32),
        pltpu.VMEM((CHUNK,), jnp.int32),
        pltpu.VMEM((CHUNK,), jnp.int32),
        pltpu.VMEM((CHUNK, D), jnp.float32),
        pltpu.SemaphoreType.DMA,
    ]
    if with_deg:
        scratch_types.append(pltpu.VMEM_SHARED((N,), jnp.float32))
        scratch_types.append(pltpu.VMEM((CHUNK,), jnp.float32))
        scratch_types.append(pltpu.VMEM((CHUNK,), jnp.float32))

    _row_chunks = []
    _o = 0
    while _o < ROWS_PER_SUB:
        _n = min(CHUNK, ROWS_PER_SUB - _o)
        _row_chunks.append((_o, _n))
        _o += _n

    @functools.partial(pl.kernel, mesh=mesh, out_type=out_type,
                       scratch_types=scratch_types)
    def agg(*refs):
        if with_deg:
            (p_hbm, src_hbm, dst_hbm,
             acc_out, deg_out,
             acc_sh, idx_v, didx_v, rows_v, sem, deg_sh, ones_v, zv) = refs
        else:
            (p_hbm, src_hbm, dst_hbm,
             acc_out,
             acc_sh, idx_v, didx_v, rows_v, sem) = refs

        cid = lax.axis_index("core")
        sid = lax.axis_index("sub")
        wid = sid * NUM_CORES + cid
        r0 = pl.multiple_of(sid * ROWS_PER_SUB, 8)
        t0 = NUM_SUBCORES * ROWS_PER_SUB

        z16 = jnp.zeros((16,), jnp.float32)

        def zrow(i, carry):
            for j in range(D // 16):
                rows_v[i, pl.ds(j * 16, 16)] = z16
            return carry

        lax.fori_loop(0, CHUNK, zrow, 0)
        for (o, n) in _row_chunks:
            pltpu.sync_copy(rows_v.at[pl.ds(0, n)],
                            acc_sh.at[pl.ds(r0 + o, n)])
        if with_deg:
            for j in range(CHUNK // 16):
                ones_v[pl.ds(j * 16, 16)] = jnp.full((16,), 1.0, jnp.float32)
                zv[pl.ds(j * 16, 16)] = z16
            for k in range(ROWS_PER_SUB // CHUNK):
                pltpu.sync_copy(zv, deg_sh.at[pl.ds(r0 + k * CHUNK, CHUNK)])
            rem = ROWS_PER_SUB % CHUNK
            if rem:
                pltpu.sync_copy(zv.at[pl.ds(0, rem)],
                                deg_sh.at[pl.ds(r0 + (ROWS_PER_SUB // CHUNK) * CHUNK, rem)])

        @pl.when(sid == 0)
        def _zero_tail():
            pltpu.sync_copy(rows_v.at[pl.ds(0, ROW_TAIL)],
                            acc_sh.at[pl.ds(t0, ROW_TAIL)])
            if with_deg:
                pltpu.sync_copy(zv.at[pl.ds(0, ROW_TAIL)],
                                deg_sh.at[pl.ds(t0, ROW_TAIL)])

        plsc.subcore_barrier()

        ebase = wid * E_PER_W

        def step(t, carry):
            off = pl.multiple_of(ebase + t * CHUNK, 8)
            pltpu.sync_copy(src_hbm.at[pl.ds(off, CHUNK)], idx_v)
            pltpu.async_copy(p_hbm.at[idx_v], rows_v, sem).wait()
            pltpu.sync_copy(dst_hbm.at[pl.ds(off, CHUNK)], didx_v)
            pltpu.sync_copy(rows_v, acc_sh.at[didx_v], add=True)
            if with_deg:
                pltpu.sync_copy(ones_v, deg_sh.at[didx_v], add=True)
            return carry

        lax.fori_loop(0, NCHUNK, step, 0)

        plsc.subcore_barrier()

        for (o, n) in _row_chunks:
            pltpu.sync_copy(acc_sh.at[pl.ds(r0 + o, n)],
                            rows_v.at[pl.ds(0, n)])
            pltpu.sync_copy(rows_v.at[pl.ds(0, n)],
                            acc_out.at[cid, pl.ds(r0 + o, n)])
        if with_deg:
            for k in range(ROWS_PER_SUB // CHUNK):
                db = pl.multiple_of(cid * N + r0 + k * CHUNK, 8)
                pltpu.sync_copy(deg_sh.at[pl.ds(r0 + k * CHUNK, CHUNK)], zv)
                pltpu.sync_copy(zv, deg_out.at[pl.ds(db, CHUNK)])
            rem = ROWS_PER_SUB % CHUNK
            if rem:
                s0 = r0 + (ROWS_PER_SUB // CHUNK) * CHUNK
                db = pl.multiple_of(cid * N + s0, 8)
                pltpu.sync_copy(deg_sh.at[pl.ds(s0, rem)], zv.at[pl.ds(0, rem)])
                pltpu.sync_copy(zv.at[pl.ds(0, rem)], deg_out.at[pl.ds(db, rem)])

        @pl.when(sid == 0)
        def _write_tail():
            pltpu.sync_copy(acc_sh.at[pl.ds(t0, ROW_TAIL)],
                            rows_v.at[pl.ds(0, ROW_TAIL)])
            pltpu.sync_copy(rows_v.at[pl.ds(0, ROW_TAIL)],
                            acc_out.at[cid, pl.ds(t0, ROW_TAIL)])
            if with_deg:
                tb = pl.multiple_of(cid * N + t0, 8)
                pltpu.sync_copy(deg_sh.at[pl.ds(t0, ROW_TAIL)],
                                zv.at[pl.ds(0, ROW_TAIL)])
                pltpu.sync_copy(zv.at[pl.ds(0, ROW_TAIL)],
                                deg_out.at[pl.ds(tb, ROW_TAIL)])

    return agg


_agg_with_deg = _make_agg(True)
_agg_no_deg = _make_agg(False)


_BLK = 1000


def _dense_pre(x, W0, b0, W1):
    def body(x_ref, w0_ref, b0_ref, w1_ref, o_ref):
        h = lax.dot_general(x_ref[...], w0_ref[...], (((1,), (1,)), ((), ())),
                            preferred_element_type=jnp.float32) + b0_ref[...]
        h = _selu(h)
        o_ref[...] = lax.dot_general(h, w1_ref[...], (((1,), (1,)), ((), ())),
                                     preferred_element_type=jnp.float32)

    return pl.pallas_call(
        body,
        grid=(N // _BLK,),
        in_specs=[
            pl.BlockSpec((_BLK, D), lambda i: (i, 0)),
            pl.BlockSpec((D, D), lambda i: (0, 0)),
            pl.BlockSpec((1, D), lambda i: (0, 0)),
            pl.BlockSpec((D, D), lambda i: (0, 0)),
        ],
        out_specs=pl.BlockSpec((_BLK, D), lambda i: (i, 0)),
        out_shape=jax.ShapeDtypeStruct((N, D), jnp.float32),
    )(x, W0, b0.reshape(1, D), W1)


def _post_agg(a0, a1, d0, d1, b_in, W, b_out):
    def body(a0_ref, a1_ref, d0_ref, d1_ref, bi_ref, w_ref, bo_ref, o_ref):
        deg = jnp.maximum(d0_ref[...] + d1_ref[...], 1.0)
        agg = (a0_ref[...] + a1_ref[...]) * (1.0 / deg)
        h = _selu(agg + bi_ref[...])
        o_ref[...] = lax.dot_general(h, w_ref[...], (((1,), (1,)), ((), ())),
                                     preferred_element_type=jnp.float32) + bo_ref[...]

    return pl.pallas_call(
        body,
        grid=(N // _BLK,),
        in_specs=[
            pl.BlockSpec((_BLK, D), lambda i: (i, 0)),
            pl.BlockSpec((_BLK, D), lambda i: (i, 0)),
            pl.BlockSpec((_BLK, 1), lambda i: (i, 0)),
            pl.BlockSpec((_BLK, 1), lambda i: (i, 0)),
            pl.BlockSpec((1, D), lambda i: (0, 0)),
            pl.BlockSpec((D, D), lambda i: (0, 0)),
            pl.BlockSpec((1, D), lambda i: (0, 0)),
        ],
        out_specs=pl.BlockSpec((_BLK, D), lambda i: (i, 0)),
        out_shape=jax.ShapeDtypeStruct((N, D), jnp.float32),
    )(a0, a1, d0, d1, b_in.reshape(1, D), W, b_out.reshape(1, D))


def kernel(x, edge_index, W0, b0, W1, b1, W2, b2, W3, b3):
    src = edge_index[0]
    dst = edge_index[1]

    p1 = _dense_pre(x, W0, b0, W1)
    accs, degs = _agg_with_deg(p1, src, dst)
    d0 = degs[:N].reshape(N, 1)
    d1 = degs[N:].reshape(N, 1)
    p2 = _post_agg(accs[0], accs[1], d0, d1, b1, W2, jnp.zeros((D,), jnp.float32))
    accs2 = _agg_no_deg(p2, src, dst)
    if isinstance(accs2, (list, tuple)):
        accs2 = accs2[0]
    out = _post_agg(accs2[0], accs2[1], d0, d1, b2, W3, b3)
    return out
